# baseline (speedup 1.0000x reference)
import functools

import jax
import jax.numpy as jnp
from jax import lax
from jax.experimental import pallas as pl
from jax.experimental.pallas import tpu as pltpu
from jax.experimental.pallas import tpu_sc as plsc

B = 262144
NUM_TILES = 64
SIG_DIM = 16
NC = 2
NS = 16
L = 16
NW = NC * NS
RPW = B // NW
CH = 512
NCHUNK = RPW // CH


def _round_bf16(v):
    u = lax.bitcast_convert_type(v, jnp.uint32)
    lsb = lax.shift_right_logical(u, jnp.uint32(16)) & jnp.uint32(1)
    r = (u + jnp.uint32(0x7FFF) + lsb) & jnp.uint32(0xFFFF0000)
    return lax.bitcast_convert_type(r, jnp.float32)


def _sc_body(sigt_hbm, tsig_hbm, scorest_hbm, idx_hbm, st_v, sc_v, ix_v,
             ts_vm, ts_v):
    wid = lax.axis_index("s") * NC + lax.axis_index("c")
    base = wid * RPW
    pltpu.sync_copy(tsig_hbm, ts_vm)

    def fill_smem(j, carry):
        vv = ts_vm[pl.ds(j * L, L)]
        for l in range(L):
            ts_v[j * L + l] = vv[l]
        return carry

    lax.fori_loop(0, (NUM_TILES * SIG_DIM) // L, fill_smem, 0)

    def chunk_body(ci, carry):
        cbase = base + ci * CH
        pltpu.sync_copy(sigt_hbm.at[:, pl.ds(cbase, CH)], st_v)

        def group_body(g, carry2):
            col0 = g * L
            svs = [_round_bf16(st_v[k, pl.ds(col0, L)]) for k in range(SIG_DIM)]
            best = jnp.full((L,), -jnp.inf, jnp.float32)
            bidx = jnp.zeros((L,), jnp.int32)
            for t in range(NUM_TILES):
                acc = svs[0] * ts_v[t]
                for k in range(1, SIG_DIM):
                    acc = acc + svs[k] * ts_v[k * NUM_TILES + t]
                sc_v[t, pl.ds(col0, L)] = acc
                better = acc > best
                best = jnp.where(better, acc, best)
                bidx = jnp.where(better, jnp.full((L,), t, jnp.int32), bidx)
            ix_v[pl.ds(col0, L)] = bidx
            return carry2

        lax.fori_loop(0, CH // L, group_body, 0)
        pltpu.sync_copy(sc_v, scorest_hbm.at[:, pl.ds(cbase, CH)])
        pltpu.sync_copy(ix_v, idx_hbm.at[pl.ds(cbase, CH)])
        return carry

    lax.fori_loop(0, NCHUNK, chunk_body, 0)


_sc_call = functools.partial(
    pl.kernel,
    mesh=plsc.VectorSubcoreMesh(core_axis_name="c", subcore_axis_name="s"),
    out_type=[
        jax.ShapeDtypeStruct((NUM_TILES, B), jnp.float32),
        jax.ShapeDtypeStruct((B,), jnp.int32),
    ],
    scratch_types=[
        pltpu.VMEM((SIG_DIM, CH), jnp.float32),
        pltpu.VMEM((NUM_TILES, CH), jnp.float32),
        pltpu.VMEM((CH,), jnp.int32),
        pltpu.VMEM((NUM_TILES * SIG_DIM,), jnp.float32),
        pltpu.SMEM((NUM_TILES * SIG_DIM,), jnp.float32),
    ],
)(_sc_body)


def kernel(sig, tile_signatures):
    sigt = sig.T
    tsig_flat = tile_signatures.T.reshape(-1)
    scorest, idx = _sc_call(sigt, tsig_flat)
    return scorest.T, idx


# speedup vs baseline: 8.8914x; 8.8914x over previous
import functools

import jax
import jax.numpy as jnp
from jax import lax
from jax.experimental import pallas as pl
from jax.experimental.pallas import tpu as pltpu
from jax.experimental.pallas import tpu_sc as plsc

B = 262144
NUM_TILES = 64
SIG_DIM = 16
RB = 8192

NC = 2
NS = 16
L = 16
NW = NC * NS
CH = 512
B_SC = NW * CH
B_TC = B - B_SC


def _tc_body(sigt_ref, tsig_ref, scorest_ref, idx_ref):
    st = sigt_ref[...]
    t = tsig_ref[...]
    sc = jax.lax.dot_general(
        t, st, (((1,), (0,)), ((), ())),
        preferred_element_type=jnp.float32)
    scorest_ref[...] = sc
    mx = jnp.max(sc, axis=0, keepdims=True)
    iota = jax.lax.broadcasted_iota(jnp.int32, sc.shape, 0)
    idx_ref[...] = jnp.min(jnp.where(sc == mx, iota, NUM_TILES), axis=0)


def _round_bf16(v):
    u = lax.bitcast_convert_type(v, jnp.uint32)
    lsb = lax.shift_right_logical(u, jnp.uint32(16)) & jnp.uint32(1)
    r = (u + jnp.uint32(0x7FFF) + lsb) & jnp.uint32(0xFFFF0000)
    return lax.bitcast_convert_type(r, jnp.float32)


def _sc_body(sigt_hbm, tsig_hbm, idx_hbm, st_v, ix_v, ts_vm, ts_v):
    wid = lax.axis_index("s") * NC + lax.axis_index("c")
    base = B_TC + wid * CH
    pltpu.sync_copy(tsig_hbm, ts_vm)

    def fill_smem(j, carry):
        vv = ts_vm[pl.ds(j * L, L)]
        for l in range(L):
            ts_v[j * L + l] = vv[l]
        return carry

    lax.fori_loop(0, (NUM_TILES * SIG_DIM) // L, fill_smem, 0)
    pltpu.sync_copy(sigt_hbm.at[:, pl.ds(base, CH)], st_v)

    def group_body(g, carry):
        col0 = g * L
        svs = [_round_bf16(st_v[k, pl.ds(col0, L)]) for k in range(SIG_DIM)]
        best = jnp.full((L,), -jnp.inf, jnp.float32)
        bidx = jnp.zeros((L,), jnp.int32)
        for t in range(NUM_TILES):
            acc = svs[0] * ts_v[t]
            for k in range(1, SIG_DIM):
                acc = acc + svs[k] * ts_v[k * NUM_TILES + t]
            better = acc > best
            best = jnp.where(better, acc, best)
            bidx = jnp.where(better, jnp.full((L,), t, jnp.int32), bidx)
        ix_v[pl.ds(col0, L)] = bidx
        return carry

    lax.fori_loop(0, CH // L, group_body, 0)
    pltpu.sync_copy(ix_v, idx_hbm.at[pl.ds(wid * CH, CH)])


_sc_call = functools.partial(
    pl.kernel,
    mesh=plsc.VectorSubcoreMesh(core_axis_name="c", subcore_axis_name="s"),
    out_type=[
        jax.ShapeDtypeStruct((B_SC,), jnp.int32),
    ],
    scratch_types=[
        pltpu.VMEM((SIG_DIM, CH), jnp.float32),
        pltpu.VMEM((CH,), jnp.int32),
        pltpu.VMEM((NUM_TILES * SIG_DIM,), jnp.float32),
        pltpu.SMEM((NUM_TILES * SIG_DIM,), jnp.float32),
    ],
)(_sc_body)


def kernel(sig, tile_signatures):
    sigt = sig.T
    tsig_flat = tile_signatures.T.reshape(-1)
    (sc_idx,) = _sc_call(sigt, tsig_flat)
    scorest, tc_idx = pl.pallas_call(
        _tc_body,
        grid=(B // RB,),
        in_specs=[
            pl.BlockSpec((SIG_DIM, RB), lambda i: (0, i)),
            pl.BlockSpec((NUM_TILES, SIG_DIM), lambda i: (0, 0)),
        ],
        out_specs=[
            pl.BlockSpec((NUM_TILES, RB), lambda i: (0, i)),
            pl.BlockSpec((RB,), lambda i: (i,)),
        ],
        out_shape=[
            jax.ShapeDtypeStruct((NUM_TILES, B), jnp.float32),
            jax.ShapeDtypeStruct((B,), jnp.int32),
        ],
    )(sigt, tile_signatures)
    idx = jnp.concatenate([tc_idx[:B_TC], sc_idx])
    return scorest.T, idx


# speedup vs baseline: 16.4398x; 1.8490x over previous
import jax
import jax.numpy as jnp
from jax.experimental import pallas as pl

B = 262144
NUM_TILES = 64
SIG_DIM = 16
RB = 16384


def _body(sigt_ref, tsig_ref, scorest_ref, idx_ref):
    st = sigt_ref[...]
    t = tsig_ref[...]
    sc = jax.lax.dot_general(
        t, st, (((1,), (0,)), ((), ())),
        preferred_element_type=jnp.float32)
    scorest_ref[...] = sc
    mx = jnp.max(sc, axis=0, keepdims=True)
    iota = jax.lax.broadcasted_iota(jnp.int32, sc.shape, 0)
    idx_ref[...] = jnp.min(jnp.where(sc == mx, iota, NUM_TILES), axis=0)


def kernel(sig, tile_signatures):
    sigt = sig.T
    scorest, idx = pl.pallas_call(
        _body,
        grid=(B // RB,),
        in_specs=[
            pl.BlockSpec((SIG_DIM, RB), lambda i: (0, i)),
            pl.BlockSpec((NUM_TILES, SIG_DIM), lambda i: (0, 0)),
        ],
        out_specs=[
            pl.BlockSpec((NUM_TILES, RB), lambda i: (0, i)),
            pl.BlockSpec((RB,), lambda i: (i,)),
        ],
        out_shape=[
            jax.ShapeDtypeStruct((NUM_TILES, B), jnp.float32),
            jax.ShapeDtypeStruct((B,), jnp.int32),
        ],
    )(sigt, tile_signatures)
    return scorest.T, idx


# speedup vs baseline: 18.3007x; 1.1132x over previous
import jax
import jax.numpy as jnp
from jax.experimental import pallas as pl

B = 262144
NUM_TILES = 64
SIG_DIM = 16
RB = 32768


def _body(sigt_ref, tsig_ref, scorest_ref, idx_ref):
    st = sigt_ref[...]
    t = tsig_ref[...]
    sc = jax.lax.dot_general(
        t, st, (((1,), (0,)), ((), ())),
        preferred_element_type=jnp.float32)
    scorest_ref[...] = sc
    mx = jnp.max(sc, axis=0, keepdims=True)
    iota = jax.lax.broadcasted_iota(jnp.int32, sc.shape, 0)
    idx_ref[...] = jnp.min(jnp.where(sc == mx, iota, NUM_TILES), axis=0)


def kernel(sig, tile_signatures):
    sigt = sig.T
    scorest, idx = pl.pallas_call(
        _body,
        grid=(B // RB,),
        in_specs=[
            pl.BlockSpec((SIG_DIM, RB), lambda i: (0, i)),
            pl.BlockSpec((NUM_TILES, SIG_DIM), lambda i: (0, 0)),
        ],
        out_specs=[
            pl.BlockSpec((NUM_TILES, RB), lambda i: (0, i)),
            pl.BlockSpec((RB,), lambda i: (i,)),
        ],
        out_shape=[
            jax.ShapeDtypeStruct((NUM_TILES, B), jnp.float32),
            jax.ShapeDtypeStruct((B,), jnp.int32),
        ],
    )(sigt, tile_signatures)
    return scorest.T, idx
